# baseline (speedup 1.0000x reference)
import jax
import jax.numpy as jnp
from jax import lax
from jax.experimental import pallas as pl
from jax.experimental.pallas import tpu as pltpu
from jax.experimental.pallas import tpu_sc as plsc
import functools

IDX_W = 128
NB = 512
NJ = NB // IDX_W


def _make_lookup(B, H, D):
    mesh = plsc.VectorSubcoreMesh(core_axis_name="c", subcore_axis_name="s")
    nc = mesh.num_cores
    g_dim = D // 8
    inner = (B // IDX_W) * 8 * IDX_W
    blk = NJ * 8 * IDX_W

    @functools.partial(
        pl.kernel,
        out_type=jax.ShapeDtypeStruct((H, g_dim, inner), jnp.float32),
        mesh=mesh,
        scratch_types=[
            pltpu.VMEM((H, NB), jnp.int32),
            pltpu.VMEM((NB, D), jnp.float32),
            pltpu.VMEM((NB, D), jnp.float32),
            pltpu.VMEM((g_dim, blk), jnp.float32),
            pltpu.VMEM((g_dim, blk), jnp.float32),
            pltpu.SemaphoreType.DMA,
            pltpu.SemaphoreType.DMA,
            pltpu.SemaphoreType.DMA,
            pltpu.SemaphoreType.DMA,
        ],
        compiler_params=pltpu.CompilerParams(
            use_tc_tiling_on_sc=False, needs_layout_passes=False
        ),
    )
    def lookup(tid_hbm, table_hbm, out_hbm, idxv, rows0, rows1, st0, st1,
               gsem0, gsem1, wsem0, wsem1):
        wid = lax.axis_index("s") * nc + lax.axis_index("c")
        b0 = wid * NB
        f0 = wid * blk
        i16 = lax.iota(jnp.int32, 16)

        pltpu.sync_copy(tid_hbm.at[:, pl.ds(b0, NB)], idxv)

        def fire(h, rows, sem):
            return [
                pltpu.async_copy(
                    table_hbm.at[idxv.at[h, pl.ds(j * IDX_W, IDX_W)]],
                    rows.at[pl.ds(j * IDX_W, IDX_W)],
                    sem,
                )
                for j in range(NJ)
            ]

        def transpose(rows, st):
            @pl.loop(0, NJ * 8)
            def _(m):
                j_ = m >> 3
                l0 = (m & 7) << 4
                riv = i16 + jnp.broadcast_to(j_ * IDX_W + l0, (16,))
                cbase = j_ * (8 * IDX_W) + l0
                @pl.loop(0, D, unroll=8)
                def _(d):
                    col = jnp.broadcast_to(d, (16,))
                    v = plsc.load_gather(rows, [riv, col])
                    st[d >> 3, pl.ds(cbase + ((d & 7) << 7), 16)] = v

        def writeback(h, st, sem):
            for g_ in range(g_dim):
                pltpu.async_copy(
                    st.at[g_], out_hbm.at[h, g_, pl.ds(f0, blk)], sem
                )

        def drain_wb(st, sem):
            pltpu.make_async_copy(st, out_hbm.at[0, :, pl.ds(f0, blk)], sem).wait()

        @pl.loop(0, H, step=2)
        def _(g):
            c0 = fire(g, rows0, gsem0)
            @pl.when(g > 2)
            def _():
                drain_wb(st1, wsem1)
            @pl.when(g > 0)
            def _():
                transpose(rows1, st1)
                writeback(g - 1, st1, wsem1)
            for c in c0:
                c.wait()
            c1 = fire(g + 1, rows1, gsem1)
            @pl.when(g > 0)
            def _():
                drain_wb(st0, wsem0)
            transpose(rows0, st0)
            writeback(g, st0, wsem0)
            for c in c1:
                c.wait()

        drain_wb(st1, wsem1)
        transpose(rows1, st1)
        writeback(H - 1, st1, wsem1)
        drain_wb(st0, wsem0)
        drain_wb(st1, wsem1)

    return lookup


def kernel(token_ids, weight):
    B, H = token_ids.shape
    V, D = weight.shape
    tid_t = token_ids.astype(jnp.int32).T
    o = _make_lookup(B, H, D)(tid_t, weight)
    o5 = o.reshape(H, D // 8, B // IDX_W, 8, IDX_W)
    t1 = jnp.transpose(o5, (2, 4, 0, 1, 3))
    return t1.reshape(B, H, D)


# speedup vs baseline: 1.3184x; 1.3184x over previous
import jax
import jax.numpy as jnp
from jax import lax
from jax.experimental import pallas as pl
from jax.experimental.pallas import tpu as pltpu
from jax.experimental.pallas import tpu_sc as plsc
import functools

IDX_W = 128
NB = 512
NJ = NB // IDX_W


def _make_lookup(B, H, D):
    mesh = plsc.VectorSubcoreMesh(core_axis_name="c", subcore_axis_name="s")
    nc = mesh.num_cores
    g_dim = D // 8
    inner = (B // IDX_W) * 8 * IDX_W
    blk = NJ * 8 * IDX_W

    @functools.partial(
        pl.kernel,
        out_type=jax.ShapeDtypeStruct((H, g_dim, inner), jnp.float32),
        mesh=mesh,
        scratch_types=[
            pltpu.VMEM((H, NB), jnp.int32),
            pltpu.VMEM((NB, D), jnp.float32),
            pltpu.VMEM((NB, D), jnp.float32),
            pltpu.VMEM((g_dim, blk), jnp.float32),
            pltpu.VMEM((g_dim, blk), jnp.float32),
            pltpu.SemaphoreType.DMA,
            pltpu.SemaphoreType.DMA,
            pltpu.SemaphoreType.DMA,
            pltpu.SemaphoreType.DMA,
        ],
        compiler_params=pltpu.CompilerParams(
            use_tc_tiling_on_sc=False,
            needs_layout_passes=False,
            disable_bounds_checks=True,
        ),
    )
    def lookup(tid_hbm, table_hbm, out_hbm, idxv, rows0, rows1, st0, st1,
               gsem0, gsem1, wsem0, wsem1):
        wid = lax.axis_index("s") * nc + lax.axis_index("c")
        b0 = wid * NB
        f0 = wid * blk
        i16 = lax.iota(jnp.int32, 16)

        pltpu.sync_copy(tid_hbm.at[:, pl.ds(b0, NB)], idxv)

        def fire(h, rows, sem):
            return [
                pltpu.async_copy(
                    table_hbm.at[idxv.at[h, pl.ds(j * IDX_W, IDX_W)]],
                    rows.at[pl.ds(j * IDX_W, IDX_W)],
                    sem,
                )
                for j in range(NJ)
            ]

        def transpose(rows, st):
            @pl.loop(0, NJ * 8)
            def _(m):
                j_ = m >> 3
                l0 = (m & 7) << 4
                riv = i16 + jnp.broadcast_to(j_ * IDX_W + l0, (16,))
                cbase = j_ * (8 * IDX_W) + l0
                vs = [
                    plsc.load_gather(rows, [riv, jnp.broadcast_to(d, (16,))])
                    for d in range(D)
                ]
                for d in range(D):
                    st[d >> 3, pl.ds(cbase + ((d & 7) << 7), 16)] = vs[d]

        def writeback(h, st, sem):
            for g_ in range(g_dim):
                pltpu.async_copy(
                    st.at[g_], out_hbm.at[h, g_, pl.ds(f0, blk)], sem
                )

        def drain_wb(st, sem):
            pltpu.make_async_copy(st, out_hbm.at[0, :, pl.ds(f0, blk)], sem).wait()

        @pl.loop(0, H, step=2)
        def _(g):
            c0 = fire(g, rows0, gsem0)
            @pl.when(g > 2)
            def _():
                drain_wb(st1, wsem1)
            @pl.when(g > 0)
            def _():
                transpose(rows1, st1)
                writeback(g - 1, st1, wsem1)
            for c in c0:
                c.wait()
            c1 = fire(g + 1, rows1, gsem1)
            @pl.when(g > 0)
            def _():
                drain_wb(st0, wsem0)
            transpose(rows0, st0)
            writeback(g, st0, wsem0)
            for c in c1:
                c.wait()

        drain_wb(st1, wsem1)
        transpose(rows1, st1)
        writeback(H - 1, st1, wsem1)
        drain_wb(st0, wsem0)
        drain_wb(st1, wsem1)

    return lookup


def kernel(token_ids, weight):
    B, H = token_ids.shape
    V, D = weight.shape
    tid_t = token_ids.astype(jnp.int32).T
    o = _make_lookup(B, H, D)(tid_t, weight)
    o5 = o.reshape(H, D // 8, B // IDX_W, 8, IDX_W)
    t1 = jnp.transpose(o5, (2, 4, 0, 1, 3))
    return t1.reshape(B, H, D)
